# baseline (speedup 1.0000x reference)
import functools

import jax
import jax.numpy as jnp
from jax import lax
from jax.experimental import pallas as pl
from jax.experimental.pallas import tpu as pltpu
from jax.experimental.pallas import tpu_sc as plsc

_NC = 2
_NS = 16
_NW = _NC * _NS
_L = 16
_DEPTH = 4


def _iota16():
    return lax.iota(jnp.int32, _L)


def _splat(v):
    return jnp.full((_L,), v, jnp.int32)


@functools.cache
def _build_transpose(V, D):
    NBLK = V // 128
    TAIL = V - NBLK * 128
    per_w = NBLK // _NW
    rem = NBLK - per_w * _NW
    assert per_w % _DEPTH == 0
    mesh = plsc.VectorSubcoreMesh(core_axis_name="c", subcore_axis_name="s")

    @functools.partial(
        pl.kernel,
        out_type=jax.ShapeDtypeStruct((V * D,), jnp.float32),
        mesh=mesh,
        compiler_params=pltpu.CompilerParams(use_tc_tiling_on_sc=True,
                                             needs_layout_passes=False),
        scratch_types=(
            [pltpu.VMEM((D, 128), jnp.float32)] * _DEPTH
            + [pltpu.VMEM((128 * D,), jnp.float32)] * _DEPTH
            + [pltpu.VMEM((max(TAIL * D, _L),), jnp.float32)]
            + [pltpu.SemaphoreType.DMA] * (2 * _DEPTH)
        ),
    )
    def ktrans(tblT_hbm, tail_hbm, out_hbm, *rest):
        stripes = rest[:_DEPTH]
        rows = rest[_DEPTH:2 * _DEPTH]
        tail_v = rest[2 * _DEPTH]
        gis = rest[2 * _DEPTH + 1: 2 * _DEPTH + 1 + _DEPTH]
        gos = rest[2 * _DEPTH + 1 + _DEPTH:]
        wid = lax.axis_index("s") * _NC + lax.axis_index("c")
        base = [_splat(c * _L) + _iota16() for c in range(D // _L)]

        def fire_in(w, b):
            k = wid + b * _NW
            pltpu.async_copy(tblT_hbm.at[:, pl.ds(k * 128, 128)],
                             stripes[w], gis[w])

        def wait_in(w):
            pltpu.make_async_copy(tblT_hbm.at[:, pl.ds(0, 128)],
                                  stripes[w], gis[w]).wait()

        def wait_out(w):
            pltpu.make_async_copy(rows[w], out_hbm.at[pl.ds(0, 128 * D)],
                                  gos[w]).wait()

        def transpose_block(w, b):
            k = wid + b * _NW
            sv, rv = stripes[w], rows[w]

            @plsc.parallel_loop(0, 128, unroll=8)
            def row_body(i):
                col = _splat(i)
                for c in range(D // _L):
                    v = plsc.load_gather(sv, [base[c], col])
                    rv[pl.ds(i * D + c * _L, _L)] = v

            pltpu.async_copy(rv, out_hbm.at[pl.ds(k * 128 * D, 128 * D)],
                             gos[w])

        for w in range(_DEPTH - 1):
            fire_in(w, w)

        def ring_body(j, carry):
            for w in range(_DEPTH):
                b = j * _DEPTH + w
                wait_in(w)

                @pl.when(b >= _DEPTH)
                def _():
                    wait_out(w)

                transpose_block(w, b)
                nb = b + _DEPTH - 1

                @pl.when(nb < per_w)
                def _():
                    fire_in((w + _DEPTH - 1) % _DEPTH, nb)

            return carry

        lax.fori_loop(0, per_w // _DEPTH, ring_body, 0)
        for w in range(_DEPTH):
            wait_out(w)

        if rem:
            @pl.when(wid < rem)
            def _():
                k = _NW * per_w + wid
                pltpu.sync_copy(tblT_hbm.at[:, pl.ds(k * 128, 128)],
                                stripes[0])

                @plsc.parallel_loop(0, 128, unroll=8)
                def row_body(i):
                    col = _splat(i)
                    for c in range(D // _L):
                        v = plsc.load_gather(stripes[0], [base[c], col])
                        rows[0][pl.ds(i * D + c * _L, _L)] = v

                pltpu.sync_copy(rows[0],
                                out_hbm.at[pl.ds(k * 128 * D, 128 * D)])

        if TAIL:
            @pl.when(wid == _NW - 1)
            def _():
                pltpu.sync_copy(tail_hbm, tail_v.at[pl.ds(0, TAIL * D)])
                pltpu.sync_copy(tail_v.at[pl.ds(0, TAIL * D)],
                                out_hbm.at[pl.ds(NBLK * 128 * D, TAIL * D)])

    return ktrans


@functools.cache
def _build_gather(B, N, D, V):
    BB = B // 128
    assert BB == _NW, "worker split assumes B == 128 * 32"
    assert N % _DEPTH == 0
    mesh = plsc.VectorSubcoreMesh(core_axis_name="c", subcore_axis_name="s")

    @functools.partial(
        pl.kernel,
        out_type=jax.ShapeDtypeStruct((N, D // 8, BB, 8, 128), jnp.float32),
        mesh=mesh,
        compiler_params=pltpu.CompilerParams(use_tc_tiling_on_sc=False,
                                             needs_layout_passes=False),
        scratch_types=(
            [pltpu.VMEM((N, 128), jnp.int32)]
            + [pltpu.VMEM((128, D), jnp.float32)] * _DEPTH
            + [pltpu.VMEM((D // 8, 8, 128), jnp.float32)] * _DEPTH
            + [pltpu.VMEM((N, D), jnp.float32)]
            + [pltpu.SemaphoreType.DMA] * (2 * _DEPTH)
        ),
    )
    def kgath(idx_hbm, tbl_hbm, pos_hbm, out_hbm, *rest):
        idx_v = rest[0]
        rows = rest[1:1 + _DEPTH]
        obufs = rest[1 + _DEPTH:1 + 2 * _DEPTH]
        pos_v = rest[1 + 2 * _DEPTH]
        gsems = rest[2 + 2 * _DEPTH:2 + 3 * _DEPTH]
        osems = rest[2 + 3 * _DEPTH:]
        wid = lax.axis_index("s") * _NC + lax.axis_index("c")
        pltpu.sync_copy(pos_hbm, pos_v)
        pltpu.sync_copy(idx_hbm.at[:, wid, :], idx_v)
        rowsel = [_splat(g * _L) + _iota16() for g in range(8)]

        def fire_gather(w, t):
            pltpu.async_copy(tbl_hbm.at[idx_v.at[t]], rows[w], gsems[w])

        def wait_gather(w):
            pltpu.make_async_copy(tbl_hbm.at[idx_v.at[0]], rows[w],
                                  gsems[w]).wait()

        def wait_out(w):
            pltpu.make_async_copy(obufs[w], out_hbm.at[0, :, wid],
                                  osems[w]).wait()

        def compute(w, t):
            rv, ov = rows[w], obufs[w]
            tcol = _splat(t)

            @plsc.parallel_loop(0, D, unroll=4)
            def d_body(d):
                dcol = _splat(d)
                p = plsc.load_gather(pos_v, [tcol, dcol])
                dt = d // 8
                ds_ = d % 8
                for g in range(8):
                    v = plsc.load_gather(rv, [rowsel[g], dcol])
                    ov[dt, ds_, pl.ds(g * _L, _L)] = v + p

            pltpu.async_copy(ov, out_hbm.at[t, :, wid], osems[w])

        for w in range(_DEPTH - 1):
            fire_gather(w, w)

        def ring_body(j, carry):
            for w in range(_DEPTH):
                t = j * _DEPTH + w
                wait_gather(w)

                @pl.when(t >= _DEPTH)
                def _():
                    wait_out(w)

                compute(w, t)
                nt = t + _DEPTH - 1

                @pl.when(nt < N)
                def _():
                    fire_gather((w + _DEPTH - 1) % _DEPTH, nt)

            return carry

        lax.fori_loop(0, N // _DEPTH, ring_body, 0)
        for w in range(_DEPTH):
            wait_out(w)

    return kgath


def kernel(x, table, pos_embedding):
    B, N = x.shape
    V, D = table.shape
    NBLK = V // 128
    tail = table[NBLK * 128:, :].reshape(-1)
    tbl_lin = _build_transpose(V, D)(table.T, tail)
    idx3 = x.T.reshape(N, B // 128, 128).astype(jnp.int32)
    out5 = _build_gather(B, N, D, V)(idx3, tbl_lin.reshape(V, D),
                                     pos_embedding)
    return out5.transpose(2, 4, 0, 1, 3).reshape(B, N, D)


# speedup vs baseline: 1.5545x; 1.5545x over previous
import functools

import jax
import jax.numpy as jnp
from jax import lax
from jax.experimental import pallas as pl
from jax.experimental.pallas import tpu as pltpu
from jax.experimental.pallas import tpu_sc as plsc

_NC = 2
_NS = 16
_NW = _NC * _NS
_L = 16
_DEPTH = 4


def _iota16():
    return lax.iota(jnp.int32, _L)


def _splat(v):
    return jnp.full((_L,), v, jnp.int32)


@functools.cache
def _build_transpose(V, D):
    NBLK = V // 128
    TAIL = V - NBLK * 128
    per_w = NBLK // _NW
    rem = NBLK - per_w * _NW
    assert per_w % _DEPTH == 0
    mesh = plsc.VectorSubcoreMesh(core_axis_name="c", subcore_axis_name="s")

    @functools.partial(
        pl.kernel,
        out_type=jax.ShapeDtypeStruct((V * D,), jnp.float32),
        mesh=mesh,
        compiler_params=pltpu.CompilerParams(use_tc_tiling_on_sc=True,
                                             needs_layout_passes=False),
        scratch_types=(
            [pltpu.VMEM((D, 129), jnp.float32)] * _DEPTH
            + [pltpu.VMEM((128 * D,), jnp.float32)] * _DEPTH
            + [pltpu.VMEM((max(TAIL * D, _L),), jnp.float32)]
            + [pltpu.SemaphoreType.DMA] * (2 * _DEPTH)
        ),
    )
    def ktrans(tblT_hbm, tail_hbm, out_hbm, *rest):
        stripes = rest[:_DEPTH]
        rows = rest[_DEPTH:2 * _DEPTH]
        tail_v = rest[2 * _DEPTH]
        gis = rest[2 * _DEPTH + 1: 2 * _DEPTH + 1 + _DEPTH]
        gos = rest[2 * _DEPTH + 1 + _DEPTH:]
        wid = lax.axis_index("s") * _NC + lax.axis_index("c")
        base = [_splat(c * _L) + _iota16() for c in range(D // _L)]

        def fire_in(w, b):
            k = wid + b * _NW
            pltpu.async_copy(tblT_hbm.at[:, pl.ds(k * 128, 128)],
                             stripes[w].at[:, pl.ds(0, 128)], gis[w])

        def wait_in(w):
            pltpu.make_async_copy(tblT_hbm.at[:, pl.ds(0, 128)],
                                  stripes[w].at[:, pl.ds(0, 128)],
                                  gis[w]).wait()

        def wait_out(w):
            pltpu.make_async_copy(rows[w], out_hbm.at[pl.ds(0, 128 * D)],
                                  gos[w]).wait()

        def transpose_block(w, b):
            k = wid + b * _NW
            sv, rv = stripes[w], rows[w]

            @plsc.parallel_loop(0, 128, unroll=8)
            def row_body(i):
                col = _splat(i)
                for c in range(D // _L):
                    v = plsc.load_gather(sv, [base[c], col])
                    rv[pl.ds(i * D + c * _L, _L)] = v

            pltpu.async_copy(rv, out_hbm.at[pl.ds(k * 128 * D, 128 * D)],
                             gos[w])

        for w in range(_DEPTH - 1):
            fire_in(w, w)

        def ring_body(j, carry):
            for w in range(_DEPTH):
                b = j * _DEPTH + w
                wait_in(w)

                @pl.when(b >= _DEPTH)
                def _():
                    wait_out(w)

                transpose_block(w, b)
                nb = b + _DEPTH - 1

                @pl.when(nb < per_w)
                def _():
                    fire_in((w + _DEPTH - 1) % _DEPTH, nb)

            return carry

        lax.fori_loop(0, per_w // _DEPTH, ring_body, 0)
        for w in range(_DEPTH):
            wait_out(w)

        if rem:
            @pl.when(wid < rem)
            def _():
                k = _NW * per_w + wid
                pltpu.sync_copy(tblT_hbm.at[:, pl.ds(k * 128, 128)],
                                stripes[0].at[:, pl.ds(0, 128)])

                @plsc.parallel_loop(0, 128, unroll=8)
                def row_body(i):
                    col = _splat(i)
                    for c in range(D // _L):
                        v = plsc.load_gather(stripes[0], [base[c], col])
                        rows[0][pl.ds(i * D + c * _L, _L)] = v

                pltpu.sync_copy(rows[0],
                                out_hbm.at[pl.ds(k * 128 * D, 128 * D)])

        if TAIL:
            @pl.when(wid == _NW - 1)
            def _():
                pltpu.sync_copy(tail_hbm, tail_v.at[pl.ds(0, TAIL * D)])
                pltpu.sync_copy(tail_v.at[pl.ds(0, TAIL * D)],
                                out_hbm.at[pl.ds(NBLK * 128 * D, TAIL * D)])

    return ktrans


@functools.cache
def _build_gather(B, N, D, V):
    BB = B // 128
    assert BB == _NW, "worker split assumes B == 128 * 32"
    assert N % _DEPTH == 0
    mesh = plsc.VectorSubcoreMesh(core_axis_name="c", subcore_axis_name="s")

    @functools.partial(
        pl.kernel,
        out_type=jax.ShapeDtypeStruct((N, D // 8, BB, 8, 128), jnp.float32),
        mesh=mesh,
        compiler_params=pltpu.CompilerParams(use_tc_tiling_on_sc=False,
                                             needs_layout_passes=False),
        scratch_types=(
            [pltpu.VMEM((N, 128), jnp.int32)]
            + [pltpu.VMEM((128, D), jnp.float32)] * _DEPTH
            + [pltpu.VMEM((D // 8, 8, 129), jnp.float32)] * _DEPTH
            + [pltpu.VMEM((N, D), jnp.float32)]
            + [pltpu.SemaphoreType.DMA] * (2 * _DEPTH)
        ),
    )
    def kgath(idx_hbm, tbl_hbm, pos_hbm, out_hbm, *rest):
        idx_v = rest[0]
        rows = rest[1:1 + _DEPTH]
        obufs = rest[1 + _DEPTH:1 + 2 * _DEPTH]
        pos_v = rest[1 + 2 * _DEPTH]
        gsems = rest[2 + 2 * _DEPTH:2 + 3 * _DEPTH]
        osems = rest[2 + 3 * _DEPTH:]
        wid = lax.axis_index("s") * _NC + lax.axis_index("c")
        pltpu.sync_copy(pos_hbm, pos_v)
        pltpu.sync_copy(idx_hbm.at[:, wid, :], idx_v)
        dtv = [(_splat(c * _L) + _iota16()) // 8 for c in range(D // _L)]
        dsv = [(_splat(c * _L) + _iota16()) % 8 for c in range(D // _L)]

        def fire_gather(w, t):
            pltpu.async_copy(tbl_hbm.at[idx_v.at[t]], rows[w], gsems[w])

        def wait_gather(w):
            pltpu.make_async_copy(tbl_hbm.at[idx_v.at[0]], rows[w],
                                  gsems[w]).wait()

        def wait_out(w):
            pltpu.make_async_copy(obufs[w].at[:, :, pl.ds(0, 128)],
                                  out_hbm.at[0, :, wid], osems[w]).wait()

        def compute(w, t):
            rv, ov = rows[w], obufs[w]
            pvec = [pos_v[t, pl.ds(c * _L, _L)] for c in range(D // _L)]

            @plsc.parallel_loop(0, 128, unroll=4)
            def b_body(b):
                bs = _splat(b)
                for c in range(D // _L):
                    v = rv[b, pl.ds(c * _L, _L)] + pvec[c]
                    plsc.store_scatter(ov, [dtv[c], dsv[c], bs], v)

            pltpu.async_copy(ov.at[:, :, pl.ds(0, 128)],
                             out_hbm.at[t, :, wid], osems[w])

        for w in range(_DEPTH - 1):
            fire_gather(w, w)

        def ring_body(j, carry):
            for w in range(_DEPTH):
                t = j * _DEPTH + w
                wait_gather(w)

                @pl.when(t >= _DEPTH)
                def _():
                    wait_out(w)

                compute(w, t)
                nt = t + _DEPTH - 1

                @pl.when(nt < N)
                def _():
                    fire_gather((w + _DEPTH - 1) % _DEPTH, nt)

            return carry

        lax.fori_loop(0, N // _DEPTH, ring_body, 0)
        for w in range(_DEPTH):
            wait_out(w)

    return kgath


def kernel(x, table, pos_embedding):
    B, N = x.shape
    V, D = table.shape
    NBLK = V // 128
    tail = table[NBLK * 128:, :].reshape(-1)
    tbl_lin = _build_transpose(V, D)(table.T, tail)
    idx3 = x.T.reshape(N, B // 128, 128).astype(jnp.int32)
    out5 = _build_gather(B, N, D, V)(idx3, tbl_lin.reshape(V, D),
                                     pos_embedding)
    return out5.transpose(2, 4, 0, 1, 3).reshape(B, N, D)


# speedup vs baseline: 1.5589x; 1.0028x over previous
import functools

import jax
import jax.numpy as jnp
from jax import lax
from jax.experimental import pallas as pl
from jax.experimental.pallas import tpu as pltpu
from jax.experimental.pallas import tpu_sc as plsc

_NC = 2
_NS = 16
_NW = _NC * _NS
_L = 16
_DEPTH = 4


def _iota16():
    return lax.iota(jnp.int32, _L)


def _splat(v):
    return jnp.full((_L,), v, jnp.int32)


@functools.cache
def _build_transpose(V, D):
    GRP = 512
    NGRP = V // GRP // _NW * _NW
    per_w = NGRP // _NW
    REMI = V - NGRP * GRP
    REMB = REMI // 128
    TAIL = REMI - REMB * 128
    SW = 513
    mesh = plsc.VectorSubcoreMesh(core_axis_name="c", subcore_axis_name="s")

    @functools.partial(
        pl.kernel,
        out_type=jax.ShapeDtypeStruct((V * D,), jnp.float32),
        mesh=mesh,
        compiler_params=pltpu.CompilerParams(use_tc_tiling_on_sc=True,
                                             needs_layout_passes=False),
        scratch_types=(
            [pltpu.VMEM((D, SW), jnp.float32)] * 2
            + [pltpu.VMEM((GRP // 2 * D,), jnp.float32)] * 2
            + [pltpu.VMEM((max(TAIL * D, _L),), jnp.float32)]
            + [pltpu.SemaphoreType.DMA] * 4
        ),
    )
    def ktrans(tblT_hbm, tail_hbm, out_hbm, *rest):
        slabs = rest[:2]
        rows = rest[2:4]
        tail_v = rest[4]
        gis = rest[5:7]
        gos = rest[7:9]
        wid = lax.axis_index("s") * _NC + lax.axis_index("c")
        base = [_splat(c * _L) + _iota16() for c in range(D // _L)]

        def fire_in(w, g):
            m = wid + g * _NW
            for j in range(D // 8):
                pltpu.async_copy(
                    tblT_hbm.at[pl.ds(8 * j, 8), pl.ds(m * GRP, GRP)],
                    slabs[w].at[pl.ds(8 * j, 8), pl.ds(0, GRP)], gis[w])

        def wait_in(w):
            for j in range(D // 8):
                pltpu.make_async_copy(
                    tblT_hbm.at[pl.ds(0, 8), pl.ds(0, GRP)],
                    slabs[w].at[pl.ds(0, 8), pl.ds(0, GRP)], gis[w]).wait()

        def wait_out(h):
            pltpu.make_async_copy(rows[h],
                                  out_hbm.at[pl.ds(0, GRP // 2 * D)],
                                  gos[h]).wait()

        def transpose_half(w, g, h):
            m = wid + g * _NW
            sv, rv = slabs[w], rows[h]

            @plsc.parallel_loop(0, GRP // 2, unroll=8)
            def row_body(i):
                col = _splat(h * (GRP // 2) + i)
                for c in range(D // _L):
                    v = plsc.load_gather(sv, [base[c], col])
                    rv[pl.ds(i * D + c * _L, _L)] = v

            pltpu.async_copy(
                rv,
                out_hbm.at[pl.ds((m * GRP + h * (GRP // 2)) * D,
                                 GRP // 2 * D)], gos[h])

        npair = per_w // 2
        fire_in(0, 0)

        def pair_body(g2, carry):
            gA = g2 * 2
            fire_in(1, gA + 1)
            wait_in(0)
            for h in range(2):
                @pl.when(gA > 0)
                def _():
                    wait_out(h)

                transpose_half(0, gA, h)

            @pl.when(gA + 2 < per_w)
            def _():
                fire_in(0, gA + 2)

            wait_in(1)
            for h in range(2):
                wait_out(h)
                transpose_half(1, gA + 1, h)
            return carry

        lax.fori_loop(0, npair, pair_body, 0)
        if per_w % 2:
            g_last = npair * 2
            wait_in(0)
            for h in range(2):
                wait_out(h)
                transpose_half(0, g_last, h)
        for h in range(2):
            wait_out(h)

        if REMB:
            @pl.when(wid < REMB)
            def _():
                k = NGRP * (GRP // 128) + wid
                pltpu.sync_copy(tblT_hbm.at[:, pl.ds(k * 128, 128)],
                                slabs[0].at[:, pl.ds(0, 128)])

                @plsc.parallel_loop(0, 128, unroll=8)
                def row_body(i):
                    col = _splat(i)
                    for c in range(D // _L):
                        v = plsc.load_gather(slabs[0], [base[c], col])
                        rows[0][pl.ds(i * D + c * _L, _L)] = v

                pltpu.sync_copy(rows[0].at[pl.ds(0, 128 * D)],
                                out_hbm.at[pl.ds(k * 128 * D, 128 * D)])

        if TAIL:
            @pl.when(wid == _NW - 1)
            def _():
                pltpu.sync_copy(tail_hbm, tail_v.at[pl.ds(0, TAIL * D)])
                pltpu.sync_copy(tail_v.at[pl.ds(0, TAIL * D)],
                                out_hbm.at[pl.ds((V - TAIL) * D, TAIL * D)])

    return ktrans


@functools.cache
def _build_gather(B, N, D, V):
    BB = B // 128
    assert BB == _NW, "worker split assumes B == 128 * 32"
    assert N % _DEPTH == 0
    mesh = plsc.VectorSubcoreMesh(core_axis_name="c", subcore_axis_name="s")

    @functools.partial(
        pl.kernel,
        out_type=jax.ShapeDtypeStruct((N, D // 8, BB, 8, 128), jnp.float32),
        mesh=mesh,
        compiler_params=pltpu.CompilerParams(use_tc_tiling_on_sc=False,
                                             needs_layout_passes=False),
        scratch_types=(
            [pltpu.VMEM((N, 128), jnp.int32)]
            + [pltpu.VMEM((128, D), jnp.float32)] * _DEPTH
            + [pltpu.VMEM((D // 8, 8, 129), jnp.float32)] * _DEPTH
            + [pltpu.VMEM((N, D), jnp.float32)]
            + [pltpu.SemaphoreType.DMA] * (2 * _DEPTH)
        ),
    )
    def kgath(idx_hbm, tbl_hbm, pos_hbm, out_hbm, *rest):
        idx_v = rest[0]
        rows = rest[1:1 + _DEPTH]
        obufs = rest[1 + _DEPTH:1 + 2 * _DEPTH]
        pos_v = rest[1 + 2 * _DEPTH]
        gsems = rest[2 + 2 * _DEPTH:2 + 3 * _DEPTH]
        osems = rest[2 + 3 * _DEPTH:]
        wid = lax.axis_index("s") * _NC + lax.axis_index("c")
        pltpu.sync_copy(pos_hbm, pos_v)
        pltpu.sync_copy(idx_hbm.at[:, wid, :], idx_v)
        dtv = [(_splat(c * _L) + _iota16()) // 8 for c in range(D // _L)]
        dsv = [(_splat(c * _L) + _iota16()) % 8 for c in range(D // _L)]

        def fire_gather(w, t):
            pltpu.async_copy(tbl_hbm.at[idx_v.at[t]], rows[w], gsems[w])

        def wait_gather(w):
            pltpu.make_async_copy(tbl_hbm.at[idx_v.at[0]], rows[w],
                                  gsems[w]).wait()

        def wait_out(w):
            pltpu.make_async_copy(obufs[w].at[:, :, pl.ds(0, 128)],
                                  out_hbm.at[0, :, wid], osems[w]).wait()

        def compute(w, t):
            rv, ov = rows[w], obufs[w]
            pvec = [pos_v[t, pl.ds(c * _L, _L)] for c in range(D // _L)]

            @plsc.parallel_loop(0, 128, unroll=4)
            def b_body(b):
                bs = _splat(b)
                for c in range(D // _L):
                    v = rv[b, pl.ds(c * _L, _L)] + pvec[c]
                    plsc.store_scatter(ov, [dtv[c], dsv[c], bs], v)

            pltpu.async_copy(ov.at[:, :, pl.ds(0, 128)],
                             out_hbm.at[t, :, wid], osems[w])

        for w in range(_DEPTH - 1):
            fire_gather(w, w)

        def ring_body(j, carry):
            for w in range(_DEPTH):
                t = j * _DEPTH + w
                wait_gather(w)

                @pl.when(t >= _DEPTH)
                def _():
                    wait_out(w)

                compute(w, t)
                nt = t + _DEPTH - 1

                @pl.when(nt < N)
                def _():
                    fire_gather((w + _DEPTH - 1) % _DEPTH, nt)

            return carry

        lax.fori_loop(0, N // _DEPTH, ring_body, 0)
        for w in range(_DEPTH):
            wait_out(w)

    return kgath


def kernel(x, table, pos_embedding):
    B, N = x.shape
    V, D = table.shape
    NBLK = V // 128
    tail = table[NBLK * 128:, :].reshape(-1)
    tbl_lin = _build_transpose(V, D)(table.T, tail)
    idx3 = x.T.reshape(N, B // 128, 128).astype(jnp.int32)
    out5 = _build_gather(B, N, D, V)(idx3, tbl_lin.reshape(V, D),
                                     pos_embedding)
    return out5.transpose(2, 4, 0, 1, 3).reshape(B, N, D)
